# baseline (speedup 1.0000x reference)
import functools

import jax
import jax.numpy as jnp
from jax import lax
from jax.experimental import pallas as pl
from jax.experimental.pallas import tpu as pltpu
from jax.experimental.pallas import tpu_sc as plsc

V = 1000000
D = 64
B = 16384
IN = 4 * D

NC = 2
NS = 16
NW = NC * NS
BPW = B // NW
CT = 16
T = BPW // CT
IDXW = CT * 8


def _gather_features(x_real, x_imag, myi, opi):
  mesh = plsc.VectorSubcoreMesh(core_axis_name="c", subcore_axis_name="s")

  @functools.partial(
      pl.kernel,
      mesh=mesh,
      compiler_params=pltpu.CompilerParams(use_tc_tiling_on_sc=False),
      out_type=jax.ShapeDtypeStruct((B, IN), jnp.float32),
      scratch_types=[
          pltpu.VMEM((T, IDXW), jnp.int32),
          pltpu.VMEM((T, IDXW), jnp.int32),
          pltpu.VMEM((IDXW, D), jnp.float32),
          pltpu.VMEM((IDXW, D), jnp.float32),
          pltpu.VMEM((IDXW, D), jnp.float32),
          pltpu.VMEM((IDXW, D), jnp.float32),
          pltpu.VMEM((CT, IN), jnp.float32),
          pltpu.SemaphoreType.DMA,
      ],
  )
  def k(xr, xi, my_h, op_h, out, myv, opv, b0, b1, b2, b3, acc, sem):
    wid = lax.axis_index("s") * NC + lax.axis_index("c")
    pltpu.sync_copy(my_h.at[wid], myv)
    pltpu.sync_copy(op_h.at[wid], opv)
    bufs = (b0, b1, b2, b3)

    def tile(t, carry):
      cps = [
          pltpu.async_copy(xr.at[myv.at[t]], b0, sem),
          pltpu.async_copy(xi.at[myv.at[t]], b1, sem),
          pltpu.async_copy(xr.at[opv.at[t]], b2, sem),
          pltpu.async_copy(xi.at[opv.at[t]], b3, sem),
      ]
      for cp in cps:
        cp.wait()

      def row(b, c2):
        base = b * 8
        for ci, buf in enumerate(bufs):
          for dc in range(D // 16):
            sl = pl.ds(dc * 16, 16)
            s = buf[base, sl]
            for j in range(1, 8):
              s = s + buf[base + j, sl]
            acc[b, pl.ds(ci * D + dc * 16, 16)] = s
        return c2

      lax.fori_loop(0, CT, row, 0)
      pltpu.sync_copy(acc, out.at[pl.ds(wid * BPW + t * CT, CT)])
      return carry

    lax.fori_loop(0, T, tile, 0)

  return k(x_real, x_imag, myi, opi)


def _mlp(features, w1t, b1, gamma, beta, w2t, b2, w3, b3):
  BLK = 2048

  def body(f_ref, w1_ref, b1_ref, g_ref, be_ref, w2_ref, b2_ref, w3_ref,
           b3_ref, o_ref):
    f = f_ref[...]
    h = jnp.dot(f, w1_ref[...], preferred_element_type=jnp.float32)
    h = h + b1_ref[...]
    mu = jnp.mean(h, axis=-1, keepdims=True)
    var = jnp.mean((h - mu) ** 2, axis=-1, keepdims=True)
    h = (h - mu) * lax.rsqrt(var + 1e-5) * g_ref[...] + be_ref[...]
    h = jnp.maximum(h, 0.0)
    h2 = jnp.dot(h, w2_ref[...], preferred_element_type=jnp.float32)
    h2 = jnp.maximum(h2 + b2_ref[...], 0.0)
    o_ref[...] = jnp.sum(h2 * w3_ref[...], axis=1) + b3_ref[0]

  return pl.pallas_call(
      body,
      grid=(B // BLK,),
      in_specs=[
          pl.BlockSpec((BLK, IN), lambda i: (i, 0)),
          pl.BlockSpec((IN, 256), lambda i: (0, 0)),
          pl.BlockSpec((256,), lambda i: (0,)),
          pl.BlockSpec((256,), lambda i: (0,)),
          pl.BlockSpec((256,), lambda i: (0,)),
          pl.BlockSpec((256, 64), lambda i: (0, 0)),
          pl.BlockSpec((64,), lambda i: (0,)),
          pl.BlockSpec((1, 64), lambda i: (0, 0)),
          pl.BlockSpec((1,), lambda i: (0,)),
      ],
      out_specs=pl.BlockSpec((BLK,), lambda i: (i,)),
      out_shape=jax.ShapeDtypeStruct((B,), jnp.float32),
  )(features, w1t, b1, gamma, beta, w2t, b2, w3, b3)


def kernel(x_real, x_imag, my_decks, op_decks, W1, b1, gamma, beta, W2, b2,
           W3, b3):
  myi = my_decks.astype(jnp.int32).reshape(NW, T, IDXW)
  opi = op_decks.astype(jnp.int32).reshape(NW, T, IDXW)
  feats = _gather_features(x_real, x_imag, myi, opi)
  return _mlp(feats, W1.T, b1, gamma, beta, W2.T, b2, W3, b3)


# speedup vs baseline: 1.1918x; 1.1918x over previous
import functools

import jax
import jax.numpy as jnp
from jax import lax
from jax.experimental import pallas as pl
from jax.experimental.pallas import tpu as pltpu
from jax.experimental.pallas import tpu_sc as plsc

V = 1000000
D = 64
B = 16384
IN = 4 * D

NC = 2
NS = 16
NW = NC * NS
BPW = B // NW
CT = 16
T = BPW // CT
IDXW = CT * 8


def _gather_features(x, myi, opi):
  mesh = plsc.VectorSubcoreMesh(core_axis_name="c", subcore_axis_name="s")

  @functools.partial(
      pl.kernel,
      mesh=mesh,
      out_type=jax.ShapeDtypeStruct((B, IN), jnp.float32),
      scratch_types=[
          pltpu.VMEM((T, IDXW), jnp.int32),
          pltpu.VMEM((T, IDXW), jnp.int32),
          pltpu.VMEM((IDXW, 2 * D), jnp.float32),
          pltpu.VMEM((IDXW, 2 * D), jnp.float32),
          pltpu.VMEM((CT, IN), jnp.float32),
          pltpu.SemaphoreType.DMA,
      ],
  )
  def k(xt, my_h, op_h, out, myv, opv, bm, bo, acc, sem):
    wid = lax.axis_index("s") * NC + lax.axis_index("c")
    pltpu.sync_copy(my_h.at[wid], myv)
    pltpu.sync_copy(op_h.at[wid], opv)

    def tile(t, carry):
      cps = [
          pltpu.async_copy(xt.at[myv.at[t]], bm, sem),
          pltpu.async_copy(xt.at[opv.at[t]], bo, sem),
      ]
      for cp in cps:
        cp.wait()

      def row(b, c2):
        base = b * 8
        for buf, cbase in ((bm, 0), (bo, 2 * D)):
          for dc in range(2 * D // 16):
            sl = pl.ds(dc * 16, 16)
            s = buf[base, sl]
            for j in range(1, 8):
              s = s + buf[base + j, sl]
            acc[b, pl.ds(cbase + dc * 16, 16)] = s
        return c2

      lax.fori_loop(0, CT, row, 0)
      pltpu.sync_copy(acc, out.at[pl.ds(wid * BPW + t * CT, CT)])
      return carry

    lax.fori_loop(0, T, tile, 0)

  return k(x, myi, opi)


def _mlp(features, w1t, b1, gamma, beta, w2t, b2, w3, b3):
  BLK = 2048

  def body(f_ref, w1_ref, b1_ref, g_ref, be_ref, w2_ref, b2_ref, w3_ref,
           b3_ref, o_ref):
    f = f_ref[...]
    h = jnp.dot(f, w1_ref[...], preferred_element_type=jnp.float32)
    h = h + b1_ref[...]
    mu = jnp.mean(h, axis=-1, keepdims=True)
    var = jnp.mean((h - mu) ** 2, axis=-1, keepdims=True)
    h = (h - mu) * lax.rsqrt(var + 1e-5) * g_ref[...] + be_ref[...]
    h = jnp.maximum(h, 0.0)
    h2 = jnp.dot(h, w2_ref[...], preferred_element_type=jnp.float32)
    h2 = jnp.maximum(h2 + b2_ref[...], 0.0)
    o_ref[...] = jnp.sum(h2 * w3_ref[...], axis=1) + b3_ref[0]

  return pl.pallas_call(
      body,
      grid=(B // BLK,),
      in_specs=[
          pl.BlockSpec((BLK, IN), lambda i: (i, 0)),
          pl.BlockSpec((IN, 256), lambda i: (0, 0)),
          pl.BlockSpec((256,), lambda i: (0,)),
          pl.BlockSpec((256,), lambda i: (0,)),
          pl.BlockSpec((256,), lambda i: (0,)),
          pl.BlockSpec((256, 64), lambda i: (0, 0)),
          pl.BlockSpec((64,), lambda i: (0,)),
          pl.BlockSpec((1, 64), lambda i: (0, 0)),
          pl.BlockSpec((1,), lambda i: (0,)),
      ],
      out_specs=pl.BlockSpec((BLK,), lambda i: (i,)),
      out_shape=jax.ShapeDtypeStruct((B,), jnp.float32),
  )(features, w1t, b1, gamma, beta, w2t, b2, w3, b3)


def kernel(x_real, x_imag, my_decks, op_decks, W1, b1, gamma, beta, W2, b2,
           W3, b3):
  myi = my_decks.astype(jnp.int32).reshape(NW, T, IDXW)
  opi = op_decks.astype(jnp.int32).reshape(NW, T, IDXW)
  x = jnp.concatenate([x_real, x_imag], axis=1)
  feats = _gather_features(x, myi, opi)
  return _mlp(feats, W1.T, b1, gamma, beta, W2.T, b2, W3, b3)


# speedup vs baseline: 1.2389x; 1.0396x over previous
import functools

import jax
import jax.numpy as jnp
from jax import lax
from jax.experimental import pallas as pl
from jax.experimental.pallas import tpu as pltpu
from jax.experimental.pallas import tpu_sc as plsc

V = 1000000
D = 64
B = 16384
IN = 4 * D

NC = 2
NS = 16
NW = NC * NS
BPW = B // NW
CT = 16
T = BPW // CT
IDXW = CT * 8


def _gather_features(x, myi, opi):
  mesh = plsc.VectorSubcoreMesh(core_axis_name="c", subcore_axis_name="s")

  @functools.partial(
      pl.kernel,
      mesh=mesh,
      out_type=jax.ShapeDtypeStruct((B, IN), jnp.float32),
      scratch_types=[
          pltpu.VMEM((T, IDXW), jnp.int32),
          pltpu.VMEM((T, IDXW), jnp.int32),
          pltpu.VMEM((IDXW, 2 * D), jnp.float32),
          pltpu.VMEM((IDXW, 2 * D), jnp.float32),
          pltpu.VMEM((IDXW, 2 * D), jnp.float32),
          pltpu.VMEM((IDXW, 2 * D), jnp.float32),
          pltpu.VMEM((CT, IN), jnp.float32),
          pltpu.SemaphoreType.DMA,
          pltpu.SemaphoreType.DMA,
      ],
  )
  def k(xt, my_h, op_h, out, myv, opv, bm0, bo0, bm1, bo1, acc, sem0, sem1):
    wid = lax.axis_index("s") * NC + lax.axis_index("c")
    pltpu.sync_copy(my_h.at[wid], myv)
    pltpu.sync_copy(op_h.at[wid], opv)

    def fire(t, bm, bo, sem):
      pltpu.async_copy(xt.at[myv.at[t]], bm, sem)
      pltpu.async_copy(xt.at[opv.at[t]], bo, sem)

    def drain(bm, bo, sem):
      pltpu.make_async_copy(xt.at[pl.ds(0, IDXW)], bm, sem).wait()
      pltpu.make_async_copy(xt.at[pl.ds(0, IDXW)], bo, sem).wait()

    def compute(t, bm, bo):
      def row(b, c2):
        base = b * 8
        for buf, cbase in ((bm, 0), (bo, 2 * D)):
          for dc in range(2 * D // 16):
            sl = pl.ds(dc * 16, 16)
            s = buf[base, sl]
            for j in range(1, 8):
              s = s + buf[base + j, sl]
            acc[b, pl.ds(cbase + dc * 16, 16)] = s
        return c2

      lax.fori_loop(0, CT, row, 0)
      pltpu.sync_copy(acc, out.at[pl.ds(wid * BPW + t * CT, CT)])

    fire(0, bm0, bo0, sem0)

    def pair(i, carry):
      t0 = 2 * i
      fire(t0 + 1, bm1, bo1, sem1)
      drain(bm0, bo0, sem0)
      compute(t0, bm0, bo0)

      @pl.when(i < T // 2 - 1)
      def _():
        fire(t0 + 2, bm0, bo0, sem0)

      drain(bm1, bo1, sem1)
      compute(t0 + 1, bm1, bo1)
      return carry

    lax.fori_loop(0, T // 2, pair, 0)

  return k(x, myi, opi)


def _concat_transpose(xrt, xit):
  VB = 1024

  def body(a_ref, b_ref, o_ref):
    o_ref[:, 0:D] = a_ref[...].T
    o_ref[:, D:2 * D] = b_ref[...].T

  return pl.pallas_call(
      body,
      grid=(pl.cdiv(V, VB),),
      in_specs=[
          pl.BlockSpec((D, VB), lambda i: (0, i)),
          pl.BlockSpec((D, VB), lambda i: (0, i)),
      ],
      out_specs=pl.BlockSpec((VB, 2 * D), lambda i: (i, 0)),
      out_shape=jax.ShapeDtypeStruct((V, 2 * D), jnp.float32),
  )(xrt, xit)


def _mlp(features, w1t, b1, gamma, beta, w2t, b2, w3, b3):
  BLK = 2048

  def body(f_ref, w1_ref, b1_ref, g_ref, be_ref, w2_ref, b2_ref, w3_ref,
           b3_ref, o_ref):
    f = f_ref[...]
    h = jnp.dot(f, w1_ref[...], preferred_element_type=jnp.float32)
    h = h + b1_ref[...]
    mu = jnp.mean(h, axis=-1, keepdims=True)
    var = jnp.mean((h - mu) ** 2, axis=-1, keepdims=True)
    h = (h - mu) * lax.rsqrt(var + 1e-5) * g_ref[...] + be_ref[...]
    h = jnp.maximum(h, 0.0)
    h2 = jnp.dot(h, w2_ref[...], preferred_element_type=jnp.float32)
    h2 = jnp.maximum(h2 + b2_ref[...], 0.0)
    o_ref[...] = jnp.sum(h2 * w3_ref[...], axis=1) + b3_ref[0]

  return pl.pallas_call(
      body,
      grid=(B // BLK,),
      in_specs=[
          pl.BlockSpec((BLK, IN), lambda i: (i, 0)),
          pl.BlockSpec((IN, 256), lambda i: (0, 0)),
          pl.BlockSpec((256,), lambda i: (0,)),
          pl.BlockSpec((256,), lambda i: (0,)),
          pl.BlockSpec((256,), lambda i: (0,)),
          pl.BlockSpec((256, 64), lambda i: (0, 0)),
          pl.BlockSpec((64,), lambda i: (0,)),
          pl.BlockSpec((1, 64), lambda i: (0, 0)),
          pl.BlockSpec((1,), lambda i: (0,)),
      ],
      out_specs=pl.BlockSpec((BLK,), lambda i: (i,)),
      out_shape=jax.ShapeDtypeStruct((B,), jnp.float32),
  )(features, w1t, b1, gamma, beta, w2t, b2, w3, b3)


def kernel(x_real, x_imag, my_decks, op_decks, W1, b1, gamma, beta, W2, b2,
           W3, b3):
  myi = my_decks.astype(jnp.int32).reshape(NW, T, IDXW)
  opi = op_decks.astype(jnp.int32).reshape(NW, T, IDXW)
  x = _concat_transpose(x_real.T, x_imag.T)
  feats = _gather_features(x, myi, opi)
  return _mlp(feats, W1.T, b1, gamma, beta, W2.T, b2, W3, b3)


# speedup vs baseline: 1.6045x; 1.2951x over previous
import functools

import jax
import jax.numpy as jnp
from jax import lax
from jax.experimental import pallas as pl
from jax.experimental.pallas import tpu as pltpu
from jax.experimental.pallas import tpu_sc as plsc

V = 1000000
D = 64
B = 16384
IN = 4 * D

NC = 2
NS = 16
NW = NC * NS
BPW = B // NW
CT = 16
T = BPW // CT
IDXW = CT * 8


def _gather_features(x, myi, opi):
  mesh = plsc.VectorSubcoreMesh(core_axis_name="c", subcore_axis_name="s")

  @functools.partial(
      pl.kernel,
      mesh=mesh,
      out_type=jax.ShapeDtypeStruct((B, IN), jnp.float32),
      scratch_types=[
          pltpu.VMEM((T, IDXW), jnp.int32),
          pltpu.VMEM((T, IDXW), jnp.int32),
          pltpu.VMEM((IDXW, 2 * D), jnp.float32),
          pltpu.VMEM((IDXW, 2 * D), jnp.float32),
          pltpu.VMEM((IDXW, 2 * D), jnp.float32),
          pltpu.VMEM((IDXW, 2 * D), jnp.float32),
          pltpu.VMEM((CT, IN), jnp.float32),
          pltpu.SemaphoreType.DMA,
          pltpu.SemaphoreType.DMA,
      ],
  )
  def k(xt, my_h, op_h, out, myv, opv, bm0, bo0, bm1, bo1, acc, sem0, sem1):
    wid = lax.axis_index("s") * NC + lax.axis_index("c")
    pltpu.sync_copy(my_h.at[wid], myv)
    pltpu.sync_copy(op_h.at[wid], opv)

    def fire(t, bm, bo, sem):
      pltpu.async_copy(xt.at[myv.at[t]], bm, sem)
      pltpu.async_copy(xt.at[opv.at[t]], bo, sem)

    def drain(bm, bo, sem):
      pltpu.make_async_copy(xt.at[pl.ds(0, IDXW)], bm, sem).wait()
      pltpu.make_async_copy(xt.at[pl.ds(0, IDXW)], bo, sem).wait()

    def compute(t, bm, bo):
      def row(b, c2):
        base = b * 8
        for buf, cbase in ((bm, 0), (bo, 2 * D)):
          for dc in range(2 * D // 16):
            sl = pl.ds(dc * 16, 16)
            s = buf[base, sl]
            for j in range(1, 8):
              s = s + buf[base + j, sl]
            acc[b, pl.ds(cbase + dc * 16, 16)] = s
        return c2

      lax.fori_loop(0, CT, row, 0)
      pltpu.sync_copy(acc, out.at[pl.ds(wid * BPW + t * CT, CT)])

    fire(0, bm0, bo0, sem0)

    def pair(i, carry):
      t0 = 2 * i
      fire(t0 + 1, bm1, bo1, sem1)
      drain(bm0, bo0, sem0)
      compute(t0, bm0, bo0)

      @pl.when(i < T // 2 - 1)
      def _():
        fire(t0 + 2, bm0, bo0, sem0)

      drain(bm1, bo1, sem1)
      compute(t0 + 1, bm1, bo1)
      return carry

    lax.fori_loop(0, T // 2, pair, 0)

  return k(x, myi, opi)


def _concat_transpose(xrt, xit):
  VB = 2048

  def body(a_ref, b_ref, o_ref):
    o_ref[:, 0:D] = a_ref[...].T
    o_ref[:, D:2 * D] = b_ref[...].T

  return pl.pallas_call(
      body,
      grid=(pl.cdiv(V, VB),),
      in_specs=[
          pl.BlockSpec((D, VB), lambda i: (0, i)),
          pl.BlockSpec((D, VB), lambda i: (0, i)),
      ],
      out_specs=pl.BlockSpec((VB, 2 * D), lambda i: (i, 0)),
      out_shape=jax.ShapeDtypeStruct((V, 2 * D), jnp.float32),
  )(xrt, xit)


def _mlp(features, w1t, b1, gamma, beta, w2t, b2, w3, b3):
  BLK = 2048

  def body(f_ref, w1_ref, b1_ref, g_ref, be_ref, w2_ref, b2_ref, w3_ref,
           b3_ref, o_ref):
    f = f_ref[...]
    h = jnp.dot(f, w1_ref[...], preferred_element_type=jnp.float32)
    h = h + b1_ref[...]
    mu = jnp.mean(h, axis=-1, keepdims=True)
    var = jnp.mean((h - mu) ** 2, axis=-1, keepdims=True)
    h = (h - mu) * lax.rsqrt(var + 1e-5) * g_ref[...] + be_ref[...]
    h = jnp.maximum(h, 0.0)
    h2 = jnp.dot(h, w2_ref[...], preferred_element_type=jnp.float32)
    h2 = jnp.maximum(h2 + b2_ref[...], 0.0)
    o_ref[...] = jnp.sum(h2 * w3_ref[...], axis=1) + b3_ref[0]

  return pl.pallas_call(
      body,
      grid=(B // BLK,),
      in_specs=[
          pl.BlockSpec((BLK, IN), lambda i: (i, 0)),
          pl.BlockSpec((IN, 256), lambda i: (0, 0)),
          pl.BlockSpec((256,), lambda i: (0,)),
          pl.BlockSpec((256,), lambda i: (0,)),
          pl.BlockSpec((256,), lambda i: (0,)),
          pl.BlockSpec((256, 64), lambda i: (0, 0)),
          pl.BlockSpec((64,), lambda i: (0,)),
          pl.BlockSpec((1, 64), lambda i: (0, 0)),
          pl.BlockSpec((1,), lambda i: (0,)),
      ],
      out_specs=pl.BlockSpec((BLK,), lambda i: (i,)),
      out_shape=jax.ShapeDtypeStruct((B,), jnp.float32),
  )(features, w1t, b1, gamma, beta, w2t, b2, w3, b3)


def kernel(x_real, x_imag, my_decks, op_decks, W1, b1, gamma, beta, W2, b2,
           W3, b3):
  myi = my_decks.astype(jnp.int32).reshape(NW, T, IDXW)
  opi = op_decks.astype(jnp.int32).reshape(NW, T, IDXW)
  x = _concat_transpose(x_real.T, x_imag.T)
  feats = _gather_features(x, myi, opi)
  return _mlp(feats, W1.T, b1, gamma, beta, W2.T, b2, W3, b3)


# speedup vs baseline: 1.9214x; 1.1975x over previous
import functools

import jax
import jax.numpy as jnp
from jax import lax
from jax.experimental import pallas as pl
from jax.experimental.pallas import tpu as pltpu
from jax.experimental.pallas import tpu_sc as plsc

V = 1000000
D = 64
B = 16384
IN = 4 * D

NC = 2
NS = 16
NW = NC * NS
BPW = B // NW
CT = 16
T = BPW // CT
IDXW = CT * 8


def _gather_features(x, myi, opi):
  mesh = plsc.VectorSubcoreMesh(core_axis_name="c", subcore_axis_name="s")

  @functools.partial(
      pl.kernel,
      mesh=mesh,
      out_type=jax.ShapeDtypeStruct((B, IN), jnp.float32),
      scratch_types=[
          pltpu.VMEM((T, IDXW), jnp.int32),
          pltpu.VMEM((T, IDXW), jnp.int32),
          pltpu.VMEM((IDXW, 2 * D), jnp.float32),
          pltpu.VMEM((IDXW, 2 * D), jnp.float32),
          pltpu.VMEM((IDXW, 2 * D), jnp.float32),
          pltpu.VMEM((IDXW, 2 * D), jnp.float32),
          pltpu.VMEM((CT, IN), jnp.float32),
          pltpu.SemaphoreType.DMA,
          pltpu.SemaphoreType.DMA,
      ],
  )
  def k(xt, my_h, op_h, out, myv, opv, bm0, bo0, bm1, bo1, acc, sem0, sem1):
    wid = lax.axis_index("s") * NC + lax.axis_index("c")
    pltpu.sync_copy(my_h.at[wid], myv)
    pltpu.sync_copy(op_h.at[wid], opv)

    def fire(t, bm, bo, sem):
      pltpu.async_copy(xt.at[myv.at[t]], bm, sem)
      pltpu.async_copy(xt.at[opv.at[t]], bo, sem)

    def drain(bm, bo, sem):
      pltpu.make_async_copy(xt.at[pl.ds(0, IDXW)], bm, sem).wait()
      pltpu.make_async_copy(xt.at[pl.ds(0, IDXW)], bo, sem).wait()

    def compute(t, bm, bo):
      def row(b, c2):
        base = b * 8
        for buf, cbase in ((bm, 0), (bo, 2 * D)):
          for dc in range(2 * D // 16):
            sl = pl.ds(dc * 16, 16)
            s = buf[base, sl]
            for j in range(1, 8):
              s = s + buf[base + j, sl]
            acc[b, pl.ds(cbase + dc * 16, 16)] = s
        return c2

      lax.fori_loop(0, CT, row, 0)
      pltpu.sync_copy(acc, out.at[pl.ds(wid * BPW + t * CT, CT)])

    fire(0, bm0, bo0, sem0)

    def pair(i, carry):
      t0 = 2 * i
      fire(t0 + 1, bm1, bo1, sem1)
      drain(bm0, bo0, sem0)
      compute(t0, bm0, bo0)

      @pl.when(i < T // 2 - 1)
      def _():
        fire(t0 + 2, bm0, bo0, sem0)

      drain(bm1, bo1, sem1)
      compute(t0 + 1, bm1, bo1)
      return carry

    lax.fori_loop(0, T // 2, pair, 0)

  return k(x, myi, opi)


def _concat_transpose(xrt, xit):
  VB = 4096

  def body(a_ref, b_ref, o_ref):
    o_ref[:, 0:D] = a_ref[...].T
    o_ref[:, D:2 * D] = b_ref[...].T

  return pl.pallas_call(
      body,
      grid=(pl.cdiv(V, VB),),
      in_specs=[
          pl.BlockSpec((D, VB), lambda i: (0, i)),
          pl.BlockSpec((D, VB), lambda i: (0, i)),
      ],
      out_specs=pl.BlockSpec((VB, 2 * D), lambda i: (i, 0)),
      out_shape=jax.ShapeDtypeStruct((V, 2 * D), jnp.float32),
  )(xrt, xit)


def _mlp(features, w1t, b1, gamma, beta, w2t, b2, w3, b3):
  BLK = 2048

  def body(f_ref, w1_ref, b1_ref, g_ref, be_ref, w2_ref, b2_ref, w3_ref,
           b3_ref, o_ref):
    f = f_ref[...]
    h = jnp.dot(f, w1_ref[...], preferred_element_type=jnp.float32)
    h = h + b1_ref[...]
    mu = jnp.mean(h, axis=-1, keepdims=True)
    var = jnp.mean((h - mu) ** 2, axis=-1, keepdims=True)
    h = (h - mu) * lax.rsqrt(var + 1e-5) * g_ref[...] + be_ref[...]
    h = jnp.maximum(h, 0.0)
    h2 = jnp.dot(h, w2_ref[...], preferred_element_type=jnp.float32)
    h2 = jnp.maximum(h2 + b2_ref[...], 0.0)
    o_ref[...] = jnp.sum(h2 * w3_ref[...], axis=1) + b3_ref[0]

  return pl.pallas_call(
      body,
      grid=(B // BLK,),
      in_specs=[
          pl.BlockSpec((BLK, IN), lambda i: (i, 0)),
          pl.BlockSpec((IN, 256), lambda i: (0, 0)),
          pl.BlockSpec((256,), lambda i: (0,)),
          pl.BlockSpec((256,), lambda i: (0,)),
          pl.BlockSpec((256,), lambda i: (0,)),
          pl.BlockSpec((256, 64), lambda i: (0, 0)),
          pl.BlockSpec((64,), lambda i: (0,)),
          pl.BlockSpec((1, 64), lambda i: (0, 0)),
          pl.BlockSpec((1,), lambda i: (0,)),
      ],
      out_specs=pl.BlockSpec((BLK,), lambda i: (i,)),
      out_shape=jax.ShapeDtypeStruct((B,), jnp.float32),
  )(features, w1t, b1, gamma, beta, w2t, b2, w3, b3)


def kernel(x_real, x_imag, my_decks, op_decks, W1, b1, gamma, beta, W2, b2,
           W3, b3):
  myi = my_decks.astype(jnp.int32).reshape(NW, T, IDXW)
  opi = op_decks.astype(jnp.int32).reshape(NW, T, IDXW)
  x = _concat_transpose(x_real.T, x_imag.T)
  feats = _gather_features(x, myi, opi)
  return _mlp(feats, W1.T, b1, gamma, beta, W2.T, b2, W3, b3)


# speedup vs baseline: 2.1329x; 1.1101x over previous
import functools

import jax
import jax.numpy as jnp
from jax import lax
from jax.experimental import pallas as pl
from jax.experimental.pallas import tpu as pltpu
from jax.experimental.pallas import tpu_sc as plsc

V = 1000000
D = 64
B = 16384
IN = 4 * D

NC = 2
NS = 16
NW = NC * NS
BPW = B // NW
CT = 16
T = BPW // CT
IDXW = CT * 8


def _gather_features(x, myi, opi):
  mesh = plsc.VectorSubcoreMesh(core_axis_name="c", subcore_axis_name="s")

  @functools.partial(
      pl.kernel,
      mesh=mesh,
      out_type=jax.ShapeDtypeStruct((B, IN), jnp.float32),
      scratch_types=[
          pltpu.VMEM((T, IDXW), jnp.int32),
          pltpu.VMEM((T, IDXW), jnp.int32),
          pltpu.VMEM((IDXW, 2 * D), jnp.float32),
          pltpu.VMEM((IDXW, 2 * D), jnp.float32),
          pltpu.VMEM((IDXW, 2 * D), jnp.float32),
          pltpu.VMEM((IDXW, 2 * D), jnp.float32),
          pltpu.VMEM((CT, IN), jnp.float32),
          pltpu.SemaphoreType.DMA,
          pltpu.SemaphoreType.DMA,
      ],
  )
  def k(xt, my_h, op_h, out, myv, opv, bm0, bo0, bm1, bo1, acc, sem0, sem1):
    wid = lax.axis_index("s") * NC + lax.axis_index("c")
    pltpu.sync_copy(my_h.at[wid], myv)
    pltpu.sync_copy(op_h.at[wid], opv)

    def fire(t, bm, bo, sem):
      pltpu.async_copy(xt.at[myv.at[t]], bm, sem)
      pltpu.async_copy(xt.at[opv.at[t]], bo, sem)

    def drain(bm, bo, sem):
      pltpu.make_async_copy(xt.at[pl.ds(0, IDXW)], bm, sem).wait()
      pltpu.make_async_copy(xt.at[pl.ds(0, IDXW)], bo, sem).wait()

    def compute(t, bm, bo):
      def row(b, c2):
        base = b * 8
        for buf, cbase in ((bm, 0), (bo, 2 * D)):
          for dc in range(2 * D // 16):
            sl = pl.ds(dc * 16, 16)
            s = buf[base, sl]
            for j in range(1, 8):
              s = s + buf[base + j, sl]
            acc[b, pl.ds(cbase + dc * 16, 16)] = s
        return c2

      lax.fori_loop(0, CT, row, 0)
      pltpu.sync_copy(acc, out.at[pl.ds(wid * BPW + t * CT, CT)])

    fire(0, bm0, bo0, sem0)

    def pair(i, carry):
      t0 = 2 * i
      fire(t0 + 1, bm1, bo1, sem1)
      drain(bm0, bo0, sem0)
      compute(t0, bm0, bo0)

      @pl.when(i < T // 2 - 1)
      def _():
        fire(t0 + 2, bm0, bo0, sem0)

      drain(bm1, bo1, sem1)
      compute(t0 + 1, bm1, bo1)
      return carry

    lax.fori_loop(0, T // 2, pair, 0)

  return k(x, myi, opi)


def _concat_transpose(xrt, xit):
  VB = 8192

  def body(a_ref, b_ref, o_ref):
    o_ref[:, 0:D] = a_ref[...].T
    o_ref[:, D:2 * D] = b_ref[...].T

  return pl.pallas_call(
      body,
      grid=(pl.cdiv(V, VB),),
      in_specs=[
          pl.BlockSpec((D, VB), lambda i: (0, i)),
          pl.BlockSpec((D, VB), lambda i: (0, i)),
      ],
      out_specs=pl.BlockSpec((VB, 2 * D), lambda i: (i, 0)),
      out_shape=jax.ShapeDtypeStruct((V, 2 * D), jnp.float32),
  )(xrt, xit)


def _mlp(features, w1t, b1, gamma, beta, w2t, b2, w3, b3):
  BLK = 2048

  def body(f_ref, w1_ref, b1_ref, g_ref, be_ref, w2_ref, b2_ref, w3_ref,
           b3_ref, o_ref):
    f = f_ref[...]
    h = jnp.dot(f, w1_ref[...], preferred_element_type=jnp.float32)
    h = h + b1_ref[...]
    mu = jnp.mean(h, axis=-1, keepdims=True)
    var = jnp.mean((h - mu) ** 2, axis=-1, keepdims=True)
    h = (h - mu) * lax.rsqrt(var + 1e-5) * g_ref[...] + be_ref[...]
    h = jnp.maximum(h, 0.0)
    h2 = jnp.dot(h, w2_ref[...], preferred_element_type=jnp.float32)
    h2 = jnp.maximum(h2 + b2_ref[...], 0.0)
    o_ref[...] = jnp.sum(h2 * w3_ref[...], axis=1) + b3_ref[0]

  return pl.pallas_call(
      body,
      grid=(B // BLK,),
      in_specs=[
          pl.BlockSpec((BLK, IN), lambda i: (i, 0)),
          pl.BlockSpec((IN, 256), lambda i: (0, 0)),
          pl.BlockSpec((256,), lambda i: (0,)),
          pl.BlockSpec((256,), lambda i: (0,)),
          pl.BlockSpec((256,), lambda i: (0,)),
          pl.BlockSpec((256, 64), lambda i: (0, 0)),
          pl.BlockSpec((64,), lambda i: (0,)),
          pl.BlockSpec((1, 64), lambda i: (0, 0)),
          pl.BlockSpec((1,), lambda i: (0,)),
      ],
      out_specs=pl.BlockSpec((BLK,), lambda i: (i,)),
      out_shape=jax.ShapeDtypeStruct((B,), jnp.float32),
  )(features, w1t, b1, gamma, beta, w2t, b2, w3, b3)


def kernel(x_real, x_imag, my_decks, op_decks, W1, b1, gamma, beta, W2, b2,
           W3, b3):
  myi = my_decks.astype(jnp.int32).reshape(NW, T, IDXW)
  opi = op_decks.astype(jnp.int32).reshape(NW, T, IDXW)
  x = _concat_transpose(x_real.T, x_imag.T)
  feats = _gather_features(x, myi, opi)
  return _mlp(feats, W1.T, b1, gamma, beta, W2.T, b2, W3, b3)


# speedup vs baseline: 2.2444x; 1.0523x over previous
import functools

import jax
import jax.numpy as jnp
from jax import lax
from jax.experimental import pallas as pl
from jax.experimental.pallas import tpu as pltpu
from jax.experimental.pallas import tpu_sc as plsc

V = 1000000
D = 64
B = 16384
IN = 4 * D

NC = 2
NS = 16
NW = NC * NS
BPW = B // NW
CT = 16
T = BPW // CT
IDXW = CT * 8


def _gather_features(x, myi, opi):
  mesh = plsc.VectorSubcoreMesh(core_axis_name="c", subcore_axis_name="s")

  @functools.partial(
      pl.kernel,
      mesh=mesh,
      out_type=jax.ShapeDtypeStruct((B, IN), jnp.float32),
      scratch_types=[
          pltpu.VMEM((T, IDXW), jnp.int32),
          pltpu.VMEM((T, IDXW), jnp.int32),
          pltpu.VMEM((IDXW, 2 * D), jnp.float32),
          pltpu.VMEM((IDXW, 2 * D), jnp.float32),
          pltpu.VMEM((IDXW, 2 * D), jnp.float32),
          pltpu.VMEM((IDXW, 2 * D), jnp.float32),
          pltpu.VMEM((CT, IN), jnp.float32),
          pltpu.SemaphoreType.DMA,
          pltpu.SemaphoreType.DMA,
      ],
  )
  def k(xt, my_h, op_h, out, myv, opv, bm0, bo0, bm1, bo1, acc, sem0, sem1):
    wid = lax.axis_index("s") * NC + lax.axis_index("c")
    pltpu.sync_copy(my_h.at[wid], myv)
    pltpu.sync_copy(op_h.at[wid], opv)

    def fire(t, bm, bo, sem):
      pltpu.async_copy(xt.at[myv.at[t]], bm, sem)
      pltpu.async_copy(xt.at[opv.at[t]], bo, sem)

    def drain(bm, bo, sem):
      pltpu.make_async_copy(xt.at[pl.ds(0, IDXW)], bm, sem).wait()
      pltpu.make_async_copy(xt.at[pl.ds(0, IDXW)], bo, sem).wait()

    def compute(t, bm, bo):
      def row(b, c2):
        base = b * 8
        for buf, cbase in ((bm, 0), (bo, 2 * D)):
          for dc in range(2 * D // 16):
            sl = pl.ds(dc * 16, 16)
            s = buf[base, sl]
            for j in range(1, 8):
              s = s + buf[base + j, sl]
            acc[b, pl.ds(cbase + dc * 16, 16)] = s
        return c2

      lax.fori_loop(0, CT, row, 0)
      pltpu.sync_copy(acc, out.at[pl.ds(wid * BPW + t * CT, CT)])

    fire(0, bm0, bo0, sem0)

    def pair(i, carry):
      t0 = 2 * i
      fire(t0 + 1, bm1, bo1, sem1)
      drain(bm0, bo0, sem0)
      compute(t0, bm0, bo0)

      @pl.when(i < T // 2 - 1)
      def _():
        fire(t0 + 2, bm0, bo0, sem0)

      drain(bm1, bo1, sem1)
      compute(t0 + 1, bm1, bo1)
      return carry

    lax.fori_loop(0, T // 2, pair, 0)

  return k(x, myi, opi)


def _concat_transpose(xrt, xit):
  VB = 16384

  def body(a_ref, b_ref, o_ref):
    o_ref[:, 0:D] = a_ref[...].T
    o_ref[:, D:2 * D] = b_ref[...].T

  return pl.pallas_call(
      body,
      grid=(pl.cdiv(V, VB),),
      in_specs=[
          pl.BlockSpec((D, VB), lambda i: (0, i)),
          pl.BlockSpec((D, VB), lambda i: (0, i)),
      ],
      out_specs=pl.BlockSpec((VB, 2 * D), lambda i: (i, 0)),
      out_shape=jax.ShapeDtypeStruct((V, 2 * D), jnp.float32),
  )(xrt, xit)


def _mlp(features, w1t, b1, gamma, beta, w2t, b2, w3, b3):
  BLK = 2048

  def body(f_ref, w1_ref, b1_ref, g_ref, be_ref, w2_ref, b2_ref, w3_ref,
           b3_ref, o_ref):
    f = f_ref[...]
    h = jnp.dot(f, w1_ref[...], preferred_element_type=jnp.float32)
    h = h + b1_ref[...]
    mu = jnp.mean(h, axis=-1, keepdims=True)
    var = jnp.mean((h - mu) ** 2, axis=-1, keepdims=True)
    h = (h - mu) * lax.rsqrt(var + 1e-5) * g_ref[...] + be_ref[...]
    h = jnp.maximum(h, 0.0)
    h2 = jnp.dot(h, w2_ref[...], preferred_element_type=jnp.float32)
    h2 = jnp.maximum(h2 + b2_ref[...], 0.0)
    o_ref[...] = jnp.sum(h2 * w3_ref[...], axis=1) + b3_ref[0]

  return pl.pallas_call(
      body,
      grid=(B // BLK,),
      in_specs=[
          pl.BlockSpec((BLK, IN), lambda i: (i, 0)),
          pl.BlockSpec((IN, 256), lambda i: (0, 0)),
          pl.BlockSpec((256,), lambda i: (0,)),
          pl.BlockSpec((256,), lambda i: (0,)),
          pl.BlockSpec((256,), lambda i: (0,)),
          pl.BlockSpec((256, 64), lambda i: (0, 0)),
          pl.BlockSpec((64,), lambda i: (0,)),
          pl.BlockSpec((1, 64), lambda i: (0, 0)),
          pl.BlockSpec((1,), lambda i: (0,)),
      ],
      out_specs=pl.BlockSpec((BLK,), lambda i: (i,)),
      out_shape=jax.ShapeDtypeStruct((B,), jnp.float32),
  )(features, w1t, b1, gamma, beta, w2t, b2, w3, b3)


def kernel(x_real, x_imag, my_decks, op_decks, W1, b1, gamma, beta, W2, b2,
           W3, b3):
  myi = my_decks.astype(jnp.int32).reshape(NW, T, IDXW)
  opi = op_decks.astype(jnp.int32).reshape(NW, T, IDXW)
  x = _concat_transpose(x_real.T, x_imag.T)
  feats = _gather_features(x, myi, opi)
  return _mlp(feats, W1.T, b1, gamma, beta, W2.T, b2, W3, b3)
